# baseline (speedup 1.0000x reference)
import functools

import jax
import jax.numpy as jnp
from jax import lax
from jax.experimental import pallas as pl
from jax.experimental.pallas import tpu as pltpu
from jax.experimental.pallas import tpu_sc as plsc

N = 10000
E = 320000
D = 128
DE = 16

NC = 2
NS = 16
NW = NC * NS
EPW = E // NW
K = 80
NCHUNK = EPW // K
RPT = 624
TAIL = N - NS * RPT
ZR = 208

_mesh = plsc.VectorSubcoreMesh(core_axis_name="c", subcore_axis_name="s")


def _edge_prepass(edge_attr, dst, zer):
    NSP = 2
    NFP = NCHUNK // NSP
    scratch = [pltpu.VMEM_SHARED((N, D), jnp.float32)]
    scratch += [pltpu.VMEM((K,), jnp.int32) for _ in range(NSP)]
    scratch += [pltpu.VMEM((K, DE), jnp.float32) for _ in range(NSP)]
    scratch += [pltpu.VMEM((K, D), jnp.float32) for _ in range(NSP)]
    scratch += [pltpu.SemaphoreType.DMA for _ in range(3 * NSP)]

    @functools.partial(
        pl.kernel,
        out_type=jax.ShapeDtypeStruct((NC, N, D), jnp.float32),
        mesh=_mesh,
        scratch_types=scratch,
    )
    def k(ea_h, dst_h, zer_h, acc_h, acc_sh, *rb):
        dstb = rb[:NSP]
        earows = rb[NSP:2 * NSP]
        stage = rb[2 * NSP:3 * NSP]
        isem = rb[3 * NSP:4 * NSP]
        esem = rb[4 * NSP:5 * NSP]
        ssem = rb[5 * NSP:]
        c = lax.axis_index("c")
        s = lax.axis_index("s")
        w = s * NC + c
        base0 = w * EPW

        for b in range(NSP):
            @pl.loop(0, K)
            def _(r, b=b):
                stage[b][r, pl.ds(DE, DE)] = jnp.ones((DE,), jnp.float32)

                @pl.loop(2 * DE, D, step=16)
                def _(j, b=b):
                    stage[b][r, pl.ds(j, 16)] = jnp.zeros((16,), jnp.float32)

        pltpu.sync_copy(zer_h, acc_sh.at[pl.ds(s * RPT, RPT), :])

        @pl.when(s == NS - 1)
        def _():
            pltpu.sync_copy(zer_h.at[pl.ds(0, TAIL), :],
                            acc_sh.at[pl.ds(NS * RPT, TAIL), :])
        plsc.subcore_barrier()

        def start_loads(j, b):
            base = pl.multiple_of(base0 + j * K, 8)
            pltpu.async_copy(dst_h.at[pl.ds(base, K)], dstb[b], isem[b])
            pltpu.async_copy(ea_h.at[pl.ds(base, K), :], earows[b], esem[b])

        def wait_loads(j, b):
            base = pl.multiple_of(base0 + j * K, 8)
            pltpu.make_async_copy(dst_h.at[pl.ds(base, K)], dstb[b],
                                  isem[b]).wait()
            pltpu.make_async_copy(ea_h.at[pl.ds(base, K), :], earows[b],
                                  esem[b]).wait()

        def start_scatter(j, b):
            pltpu.async_copy(stage[b], acc_sh.at[dstb[b]], ssem[b], add=True)

        def wait_scatter(j, b):
            pltpu.make_async_copy(stage[b], acc_sh.at[dstb[b]],
                                  ssem[b]).wait()

        for b in range(NSP):
            start_loads(b, b)

        @pl.loop(0, NFP)
        def _(j2):
            j0 = j2 * NSP
            for b in range(NSP):
                j = j0 + b
                wait_loads(j, b)

                @pl.loop(0, K)
                def _(r, b=b):
                    stage[b][r, pl.ds(0, DE)] = earows[b][r, pl.ds(0, DE)]

                start_scatter(j, b)

                @pl.when(j + NSP < NCHUNK)
                def _():
                    wait_scatter(j, b)
                    start_loads(j + NSP, b)

        for j in range(NFP * NSP, NCHUNK):
            b = j % NSP
            wait_loads(j, b)

            @pl.loop(0, K)
            def _(r, b=b):
                stage[b][r, pl.ds(0, DE)] = earows[b][r, pl.ds(0, DE)]

            start_scatter(j, b)
        for j in range(NCHUNK - NSP, NCHUNK):
            wait_scatter(j, j % NSP)

        plsc.subcore_barrier()
        r0 = s * RPT
        pltpu.sync_copy(acc_sh.at[pl.ds(r0, RPT), :], acc_h.at[c, pl.ds(r0, RPT), :])

        @pl.when(s == NS - 1)
        def _():
            t0 = NS * RPT
            pltpu.sync_copy(acc_sh.at[pl.ds(t0, TAIL), :],
                            acc_h.at[c, pl.ds(t0, TAIL), :])

    return k(edge_attr, dst, zer)


KA = 80
NCHUNKA = EPW // KA
NSLOT = 2
NFULL = NCHUNKA // NSLOT


def _edge_aggregate(table, src3, dst1, zer):
    dt_ = table.dtype
    scratch = [
        pltpu.VMEM_SHARED((N, D), dt_),
        pltpu.VMEM((NCHUNKA, KA), jnp.int32),
    ]
    scratch += [pltpu.VMEM((KA,), jnp.int32) for _ in range(NSLOT)]
    scratch += [pltpu.VMEM((KA, D), dt_) for _ in range(NSLOT)]
    scratch += [pltpu.SemaphoreType.DMA for _ in range(3 * NSLOT)]

    @functools.partial(
        pl.kernel,
        out_type=jax.ShapeDtypeStruct((NC, N, D), dt_),
        mesh=_mesh,
        scratch_types=scratch,
    )
    def k(table_h, src_h, dst_h, zer_h, out_h, agg_sh, srca, *rb):
        dstb = rb[:NSLOT]
        rows = rb[NSLOT:2 * NSLOT]
        gsem = rb[2 * NSLOT:3 * NSLOT]
        ssem = rb[3 * NSLOT:4 * NSLOT]
        isem = rb[4 * NSLOT:]
        c = lax.axis_index("c")
        s = lax.axis_index("s")
        w = s * NC + c
        r0 = s * RPT
        base0 = w * EPW

        pltpu.sync_copy(src_h.at[w], srca)
        pltpu.sync_copy(zer_h, agg_sh.at[pl.ds(r0, RPT), :])

        @pl.when(s == NS - 1)
        def _():
            pltpu.sync_copy(zer_h.at[pl.ds(0, TAIL), :],
                            agg_sh.at[pl.ds(NS * RPT, TAIL), :])
        plsc.subcore_barrier()

        def start_dst(j, b):
            base = pl.multiple_of(base0 + j * KA, 8)
            pltpu.async_copy(dst_h.at[pl.ds(base, KA)], dstb[b], isem[b])

        def wait_dst(j, b):
            base = pl.multiple_of(base0 + j * KA, 8)
            pltpu.make_async_copy(dst_h.at[pl.ds(base, KA)], dstb[b],
                                  isem[b]).wait()

        def start_gather(j, b):
            pltpu.async_copy(table_h.at[srca.at[j]], rows[b], gsem[b])

        def wait_gather(j, b):
            pltpu.make_async_copy(table_h.at[srca.at[j]], rows[b],
                                  gsem[b]).wait()

        def start_scatter(j, b):
            pltpu.async_copy(rows[b], agg_sh.at[dstb[b]], ssem[b], add=True)

        def wait_scatter(j, b):
            pltpu.make_async_copy(rows[b], agg_sh.at[dstb[b]],
                                  ssem[b]).wait()

        for b in range(NSLOT):
            start_dst(b, b)
            start_gather(b, b)

        @pl.loop(0, NFULL)
        def _(j2):
            j0 = j2 * NSLOT
            for b in range(NSLOT):
                j = j0 + b
                wait_gather(j, b)
                wait_dst(j, b)
                start_scatter(j, b)

                @pl.when(j + NSLOT < NCHUNKA)
                def _():
                    wait_scatter(j, b)
                    start_gather(j + NSLOT, b)
                    start_dst(j + NSLOT, b)

        for j in range(NFULL * NSLOT, NCHUNKA):
            wait_gather(j, j % NSLOT)
            wait_dst(j, j % NSLOT)
            start_scatter(j, j % NSLOT)
        for j in range(NCHUNKA - NSLOT, NCHUNKA):
            wait_scatter(j, j % NSLOT)

        plsc.subcore_barrier()
        pltpu.sync_copy(agg_sh.at[pl.ds(r0, RPT), :], out_h.at[c, pl.ds(r0, RPT), :])

        @pl.when(s == NS - 1)
        def _():
            t0 = NS * RPT
            pltpu.sync_copy(agg_sh.at[pl.ds(t0, TAIL), :],
                            out_h.at[c, pl.ds(t0, TAIL), :])

    return k(table, src3, dst1, zer)


_BN = 2000
_GRID = (N // _BN,)


def _mm_stage(arrs, coefs, Wcat, emit_h):
    n_in = len(arrs)

    def body(*refs):
        in_refs = refs[:n_in]
        w_ref = refs[n_in]
        out_refs = refs[n_in + 1:]
        t = coefs[0] * in_refs[0][...]
        for cf, r in zip(coefs[1:], in_refs[1:]):
            t = t + cf * r[...]
        p = jnp.dot(t, w_ref[...], preferred_element_type=jnp.float32)
        o = 0
        if emit_h:
            out_refs[0][...] = t
            o = 1
        out_refs[o][...] = p[:, :D]
        out_refs[o + 1][...] = p[:, D:]

    outs = ([jax.ShapeDtypeStruct((N, D), jnp.float32)] * (3 if emit_h else 2))
    f = pl.pallas_call(
        body,
        grid=_GRID,
        in_specs=[pl.BlockSpec((_BN, D), lambda i: (i, 0)) for _ in range(n_in)]
        + [pl.BlockSpec((D, 2 * D), lambda i: (0, 0))],
        out_specs=[pl.BlockSpec((_BN, D), lambda i: (i, 0)) for _ in outs],
        out_shape=outs,
    )
    return f(*arrs, Wcat)


def _precompute(acc, W_edge):
    def body(acc_ref, we_ref, invd_ref, eat_ref):
        tot = acc_ref[0] + acc_ref[1]
        invd = 1.0 / jnp.maximum(tot[:, DE:DE + 1], 1.0)
        es = tot[:, 0:DE]
        eat = jnp.dot(es, we_ref[...], preferred_element_type=jnp.float32)
        invd_ref[...] = invd
        eat_ref[...] = eat * invd

    return pl.pallas_call(
        body,
        grid=_GRID,
        in_specs=[
            pl.BlockSpec((NC, _BN, D), lambda i: (0, i, 0)),
            pl.BlockSpec((DE, D), lambda i: (0, 0)),
        ],
        out_specs=[
            pl.BlockSpec((_BN, 1), lambda i: (i, 0)),
            pl.BlockSpec((_BN, D), lambda i: (i, 0)),
        ],
        out_shape=[
            jax.ShapeDtypeStruct((N, 1), jnp.float32),
            jax.ShapeDtypeStruct((N, D), jnp.float32),
        ],
    )(acc, W_edge)


def _combine_stage(hr, parts, ea_term, invd, b2, arrs, coefs, Wcat,
                   emit_k, emit_t):
    n_in = len(arrs)

    def body(*refs):
        hr_ref, p_ref, eat_ref, invd_ref, b_ref = refs[:5]
        in_refs = refs[5:5 + n_in]
        rest = refs[5 + n_in:]
        agg = (p_ref[0] + p_ref[1]) * invd_ref[...]
        kk = jnp.tanh(hr_ref[...] + agg + eat_ref[...] + b_ref[...])
        t = coefs[-1] * kk
        for cf, r in zip(coefs[:-1], in_refs):
            t = t + cf * r[...]
        if Wcat is not None:
            w_ref, rest = rest[0], rest[1:]
        o = 0
        if emit_k:
            rest[o][...] = kk
            o += 1
        if emit_t:
            rest[o][...] = t
            o += 1
        if Wcat is not None:
            p = jnp.dot(t, w_ref[...], preferred_element_type=jnp.float32)
            rest[o][...] = p[:, :D]
            rest[o + 1][...] = p[:, D:]

    bs = pl.BlockSpec((_BN, D), lambda i: (i, 0))
    in_specs = [
        bs,
        pl.BlockSpec((NC, _BN, D), lambda i: (0, i, 0)),
        bs,
        pl.BlockSpec((_BN, 1), lambda i: (i, 0)),
        pl.BlockSpec((1, D), lambda i: (0, 0)),
    ] + [bs] * n_in
    args = [hr, parts, ea_term, invd, b2] + list(arrs)
    if Wcat is not None:
        in_specs.append(pl.BlockSpec((D, 2 * D), lambda i: (0, 0)))
        args.append(Wcat)
    n_out = int(emit_k) + int(emit_t) + (2 if Wcat is not None else 0)
    outs = [jax.ShapeDtypeStruct((N, D), jnp.float32)] * n_out
    return pl.pallas_call(
        body,
        grid=_GRID,
        in_specs=in_specs,
        out_specs=[bs] * n_out,
        out_shape=outs,
    )(*args)


def kernel(x, edge_index, edge_attr, W_msg, W_edge, W_root, b):
    src = edge_index[0].astype(jnp.int32)
    dst = edge_index[1].astype(jnp.int32)
    src3 = src.reshape(NW, NCHUNKA, KA)
    zer = jnp.zeros((RPT, D), jnp.float32)
    Wcat = jnp.concatenate([W_msg, W_root], axis=1)
    b2 = b.reshape(1, D)
    dt = 1.0 / 3.0

    acc = _edge_prepass(edge_attr, dst, zer)
    invd, ea_term = _precompute(acc, W_edge)

    def cs(hr, parts, arrs, coefs, Wc, emit_k, emit_t):
        return _combine_stage(hr, parts, ea_term, invd, b2, arrs, coefs, Wc,
                              emit_k, emit_t)

    x_cur = x
    hm, hr = _mm_stage([x_cur], [1.0], Wcat, emit_h=False)
    for step in range(3):
        parts = _edge_aggregate(hm, src3, dst, zer)
        k1, hm, hr = cs(hr, parts, [x_cur], [1.0, dt / 2], Wcat, True, False)
        parts = _edge_aggregate(hm, src3, dst, zer)
        k2, hm, hr = cs(hr, parts, [x_cur], [1.0, dt / 2], Wcat, True, False)
        parts = _edge_aggregate(hm, src3, dst, zer)
        k3, hm, hr = cs(hr, parts, [x_cur], [1.0, dt], Wcat, True, False)
        parts = _edge_aggregate(hm, src3, dst, zer)
        arrs = [x_cur, k1, k2, k3]
        coefs = [1.0, dt / 6, dt / 3, dt / 3, dt / 6]
        if step < 2:
            x_cur, hm, hr = cs(hr, parts, arrs, coefs, Wcat, False, True)
        else:
            (x_cur,) = cs(hr, parts, arrs, coefs, None, False, True)
    return x_cur


# speedup vs baseline: 1.0103x; 1.0103x over previous
import functools

import jax
import jax.numpy as jnp
from jax import lax
from jax.experimental import pallas as pl
from jax.experimental.pallas import tpu as pltpu
from jax.experimental.pallas import tpu_sc as plsc

N = 10000
E = 320000
D = 128
DE = 16

NC = 2
NS = 16
NW = NC * NS
EPW = E // NW
K = 80
NCHUNK = EPW // K
RPT = 624
TAIL = N - NS * RPT
ZR = 208

_mesh = plsc.VectorSubcoreMesh(core_axis_name="c", subcore_axis_name="s")


def _edge_prepass(edge_attr, dst, zer):
    NSP = 2
    NFP = NCHUNK // NSP
    scratch = [pltpu.VMEM_SHARED((N, D), jnp.float32)]
    scratch += [pltpu.VMEM((K,), jnp.int32) for _ in range(NSP)]
    scratch += [pltpu.VMEM((K, DE), jnp.float32) for _ in range(NSP)]
    scratch += [pltpu.VMEM((K, D), jnp.float32) for _ in range(NSP)]
    scratch += [pltpu.SemaphoreType.DMA for _ in range(3 * NSP)]

    @functools.partial(
        pl.kernel,
        out_type=jax.ShapeDtypeStruct((NC, N, D), jnp.float32),
        mesh=_mesh,
        scratch_types=scratch,
    )
    def k(ea_h, dst_h, zer_h, acc_h, acc_sh, *rb):
        dstb = rb[:NSP]
        earows = rb[NSP:2 * NSP]
        stage = rb[2 * NSP:3 * NSP]
        isem = rb[3 * NSP:4 * NSP]
        esem = rb[4 * NSP:5 * NSP]
        ssem = rb[5 * NSP:]
        c = lax.axis_index("c")
        s = lax.axis_index("s")
        w = s * NC + c
        base0 = w * EPW

        for b in range(NSP):
            @pl.loop(0, K)
            def _(r, b=b):
                stage[b][r, pl.ds(DE, DE)] = jnp.ones((DE,), jnp.float32)

                @pl.loop(2 * DE, D, step=16)
                def _(j, b=b):
                    stage[b][r, pl.ds(j, 16)] = jnp.zeros((16,), jnp.float32)

        pltpu.sync_copy(zer_h, acc_sh.at[pl.ds(s * RPT, RPT), :])

        @pl.when(s == NS - 1)
        def _():
            pltpu.sync_copy(zer_h.at[pl.ds(0, TAIL), :],
                            acc_sh.at[pl.ds(NS * RPT, TAIL), :])
        plsc.subcore_barrier()

        def start_loads(j, b):
            base = pl.multiple_of(base0 + j * K, 8)
            pltpu.async_copy(dst_h.at[pl.ds(base, K)], dstb[b], isem[b])
            pltpu.async_copy(ea_h.at[pl.ds(base, K), :], earows[b], esem[b])

        def wait_loads(j, b):
            base = pl.multiple_of(base0 + j * K, 8)
            pltpu.make_async_copy(dst_h.at[pl.ds(base, K)], dstb[b],
                                  isem[b]).wait()
            pltpu.make_async_copy(ea_h.at[pl.ds(base, K), :], earows[b],
                                  esem[b]).wait()

        def start_scatter(j, b):
            pltpu.async_copy(stage[b], acc_sh.at[dstb[b]], ssem[b], add=True)

        def wait_scatter(j, b):
            pltpu.make_async_copy(stage[b], acc_sh.at[dstb[b]],
                                  ssem[b]).wait()

        for b in range(NSP):
            start_loads(b, b)

        @pl.loop(0, NFP)
        def _(j2):
            j0 = j2 * NSP
            for b in range(NSP):
                j = j0 + b
                wait_loads(j, b)

                @pl.loop(0, K)
                def _(r, b=b):
                    stage[b][r, pl.ds(0, DE)] = earows[b][r, pl.ds(0, DE)]

                start_scatter(j, b)

                @pl.when(j + NSP < NCHUNK)
                def _():
                    wait_scatter(j, b)
                    start_loads(j + NSP, b)

        for j in range(NFP * NSP, NCHUNK):
            b = j % NSP
            wait_loads(j, b)

            @pl.loop(0, K)
            def _(r, b=b):
                stage[b][r, pl.ds(0, DE)] = earows[b][r, pl.ds(0, DE)]

            start_scatter(j, b)
        for j in range(NCHUNK - NSP, NCHUNK):
            wait_scatter(j, j % NSP)

        plsc.subcore_barrier()
        r0 = s * RPT
        pltpu.sync_copy(acc_sh.at[pl.ds(r0, RPT), :], acc_h.at[c, pl.ds(r0, RPT), :])

        @pl.when(s == NS - 1)
        def _():
            t0 = NS * RPT
            pltpu.sync_copy(acc_sh.at[pl.ds(t0, TAIL), :],
                            acc_h.at[c, pl.ds(t0, TAIL), :])

    return k(edge_attr, dst, zer)


KA = 40
NCHUNKA = EPW // KA
NSLOT = 3
NFULL = NCHUNKA // NSLOT


def _edge_aggregate(table, src3, dst1, zer):
    dt_ = table.dtype
    scratch = [
        pltpu.VMEM_SHARED((N, D), dt_),
        pltpu.VMEM((NCHUNKA, KA), jnp.int32),
    ]
    scratch += [pltpu.VMEM((KA,), jnp.int32) for _ in range(NSLOT)]
    scratch += [pltpu.VMEM((KA, D), dt_) for _ in range(NSLOT)]
    scratch += [pltpu.SemaphoreType.DMA for _ in range(3 * NSLOT)]

    @functools.partial(
        pl.kernel,
        out_type=jax.ShapeDtypeStruct((NC, N, D), dt_),
        mesh=_mesh,
        scratch_types=scratch,
    )
    def k(table_h, src_h, dst_h, zer_h, out_h, agg_sh, srca, *rb):
        dstb = rb[:NSLOT]
        rows = rb[NSLOT:2 * NSLOT]
        gsem = rb[2 * NSLOT:3 * NSLOT]
        ssem = rb[3 * NSLOT:4 * NSLOT]
        isem = rb[4 * NSLOT:]
        c = lax.axis_index("c")
        s = lax.axis_index("s")
        w = s * NC + c
        r0 = s * RPT
        base0 = w * EPW

        pltpu.async_copy(src_h.at[w], srca, gsem[0])
        pltpu.async_copy(zer_h, agg_sh.at[pl.ds(r0, RPT), :], gsem[1])

        @pl.when(s == NS - 1)
        def _():
            pltpu.sync_copy(zer_h.at[pl.ds(0, TAIL), :],
                            agg_sh.at[pl.ds(NS * RPT, TAIL), :])
        pltpu.make_async_copy(src_h.at[w], srca, gsem[0]).wait()
        pltpu.make_async_copy(zer_h, agg_sh.at[pl.ds(r0, RPT), :],
                              gsem[1]).wait()
        plsc.subcore_barrier()

        def start_dst(j, b):
            base = pl.multiple_of(base0 + j * KA, 8)
            pltpu.async_copy(dst_h.at[pl.ds(base, KA)], dstb[b], isem[b])

        def wait_dst(j, b):
            base = pl.multiple_of(base0 + j * KA, 8)
            pltpu.make_async_copy(dst_h.at[pl.ds(base, KA)], dstb[b],
                                  isem[b]).wait()

        def start_gather(j, b):
            pltpu.async_copy(table_h.at[srca.at[j]], rows[b], gsem[b])

        def wait_gather(j, b):
            pltpu.make_async_copy(table_h.at[srca.at[j]], rows[b],
                                  gsem[b]).wait()

        def start_scatter(j, b):
            pltpu.async_copy(rows[b], agg_sh.at[dstb[b]], ssem[b], add=True)

        def wait_scatter(j, b):
            pltpu.make_async_copy(rows[b], agg_sh.at[dstb[b]],
                                  ssem[b]).wait()

        for b in range(NSLOT):
            start_dst(b, b)
            start_gather(b, b)

        @pl.loop(0, NFULL)
        def _(j2):
            j0 = j2 * NSLOT
            for b in range(NSLOT):
                j = j0 + b
                wait_gather(j, b)
                wait_dst(j, b)
                start_scatter(j, b)

                @pl.when(j + NSLOT < NCHUNKA)
                def _():
                    wait_scatter(j, b)
                    start_gather(j + NSLOT, b)
                    start_dst(j + NSLOT, b)

        for j in range(NFULL * NSLOT, NCHUNKA):
            wait_gather(j, j % NSLOT)
            wait_dst(j, j % NSLOT)
            start_scatter(j, j % NSLOT)
        for j in range(NCHUNKA - NSLOT, NCHUNKA):
            wait_scatter(j, j % NSLOT)

        plsc.subcore_barrier()
        pltpu.sync_copy(agg_sh.at[pl.ds(r0, RPT), :], out_h.at[c, pl.ds(r0, RPT), :])

        @pl.when(s == NS - 1)
        def _():
            t0 = NS * RPT
            pltpu.sync_copy(agg_sh.at[pl.ds(t0, TAIL), :],
                            out_h.at[c, pl.ds(t0, TAIL), :])

    return k(table, src3, dst1, zer)


_BN = 2000
_GRID = (N // _BN,)


def _mm_stage(arrs, coefs, Wcat, emit_h):
    n_in = len(arrs)

    def body(*refs):
        in_refs = refs[:n_in]
        w_ref = refs[n_in]
        out_refs = refs[n_in + 1:]
        t = coefs[0] * in_refs[0][...]
        for cf, r in zip(coefs[1:], in_refs[1:]):
            t = t + cf * r[...]
        p = jnp.dot(t, w_ref[...], preferred_element_type=jnp.float32)
        o = 0
        if emit_h:
            out_refs[0][...] = t
            o = 1
        out_refs[o][...] = p[:, :D]
        out_refs[o + 1][...] = p[:, D:]

    outs = ([jax.ShapeDtypeStruct((N, D), jnp.float32)] * (3 if emit_h else 2))
    f = pl.pallas_call(
        body,
        grid=_GRID,
        in_specs=[pl.BlockSpec((_BN, D), lambda i: (i, 0)) for _ in range(n_in)]
        + [pl.BlockSpec((D, 2 * D), lambda i: (0, 0))],
        out_specs=[pl.BlockSpec((_BN, D), lambda i: (i, 0)) for _ in outs],
        out_shape=outs,
    )
    return f(*arrs, Wcat)


def _precompute(acc, W_edge):
    def body(acc_ref, we_ref, invd_ref, eat_ref):
        tot = acc_ref[0] + acc_ref[1]
        invd = 1.0 / jnp.maximum(tot[:, DE:DE + 1], 1.0)
        es = tot[:, 0:DE]
        eat = jnp.dot(es, we_ref[...], preferred_element_type=jnp.float32)
        invd_ref[...] = invd
        eat_ref[...] = eat * invd

    return pl.pallas_call(
        body,
        grid=_GRID,
        in_specs=[
            pl.BlockSpec((NC, _BN, D), lambda i: (0, i, 0)),
            pl.BlockSpec((DE, D), lambda i: (0, 0)),
        ],
        out_specs=[
            pl.BlockSpec((_BN, 1), lambda i: (i, 0)),
            pl.BlockSpec((_BN, D), lambda i: (i, 0)),
        ],
        out_shape=[
            jax.ShapeDtypeStruct((N, 1), jnp.float32),
            jax.ShapeDtypeStruct((N, D), jnp.float32),
        ],
    )(acc, W_edge)


def _combine_stage(hr, parts, ea_term, invd, b2, arrs, coefs, Wcat,
                   emit_k, emit_t):
    n_in = len(arrs)

    def body(*refs):
        hr_ref, p_ref, eat_ref, invd_ref, b_ref = refs[:5]
        in_refs = refs[5:5 + n_in]
        rest = refs[5 + n_in:]
        agg = (p_ref[0] + p_ref[1]) * invd_ref[...]
        kk = jnp.tanh(hr_ref[...] + agg + eat_ref[...] + b_ref[...])
        t = coefs[-1] * kk
        for cf, r in zip(coefs[:-1], in_refs):
            t = t + cf * r[...]
        if Wcat is not None:
            w_ref, rest = rest[0], rest[1:]
        o = 0
        if emit_k:
            rest[o][...] = kk
            o += 1
        if emit_t:
            rest[o][...] = t
            o += 1
        if Wcat is not None:
            p = jnp.dot(t, w_ref[...], preferred_element_type=jnp.float32)
            rest[o][...] = p[:, :D]
            rest[o + 1][...] = p[:, D:]

    bs = pl.BlockSpec((_BN, D), lambda i: (i, 0))
    in_specs = [
        bs,
        pl.BlockSpec((NC, _BN, D), lambda i: (0, i, 0)),
        bs,
        pl.BlockSpec((_BN, 1), lambda i: (i, 0)),
        pl.BlockSpec((1, D), lambda i: (0, 0)),
    ] + [bs] * n_in
    args = [hr, parts, ea_term, invd, b2] + list(arrs)
    if Wcat is not None:
        in_specs.append(pl.BlockSpec((D, 2 * D), lambda i: (0, 0)))
        args.append(Wcat)
    n_out = int(emit_k) + int(emit_t) + (2 if Wcat is not None else 0)
    outs = [jax.ShapeDtypeStruct((N, D), jnp.float32)] * n_out
    return pl.pallas_call(
        body,
        grid=_GRID,
        in_specs=in_specs,
        out_specs=[bs] * n_out,
        out_shape=outs,
    )(*args)


def kernel(x, edge_index, edge_attr, W_msg, W_edge, W_root, b):
    src = edge_index[0].astype(jnp.int32)
    dst = edge_index[1].astype(jnp.int32)
    src3 = src.reshape(NW, NCHUNKA, KA)
    zer = jnp.zeros((RPT, D), jnp.float32)
    Wcat = jnp.concatenate([W_msg, W_root], axis=1)
    b2 = b.reshape(1, D)
    dt = 1.0 / 3.0

    acc = _edge_prepass(edge_attr, dst, zer)
    invd, ea_term = _precompute(acc, W_edge)

    def cs(hr, parts, arrs, coefs, Wc, emit_k, emit_t):
        return _combine_stage(hr, parts, ea_term, invd, b2, arrs, coefs, Wc,
                              emit_k, emit_t)

    x_cur = x
    hm, hr = _mm_stage([x_cur], [1.0], Wcat, emit_h=False)
    for step in range(3):
        parts = _edge_aggregate(hm, src3, dst, zer)
        k1, hm, hr = cs(hr, parts, [x_cur], [1.0, dt / 2], Wcat, True, False)
        parts = _edge_aggregate(hm, src3, dst, zer)
        k2, hm, hr = cs(hr, parts, [x_cur], [1.0, dt / 2], Wcat, True, False)
        parts = _edge_aggregate(hm, src3, dst, zer)
        k3, hm, hr = cs(hr, parts, [x_cur], [1.0, dt], Wcat, True, False)
        parts = _edge_aggregate(hm, src3, dst, zer)
        arrs = [x_cur, k1, k2, k3]
        coefs = [1.0, dt / 6, dt / 3, dt / 3, dt / 6]
        if step < 2:
            x_cur, hm, hr = cs(hr, parts, arrs, coefs, Wcat, False, True)
        else:
            (x_cur,) = cs(hr, parts, arrs, coefs, None, False, True)
    return x_cur
